# baseline (speedup 1.0000x reference)
import jax
import jax.numpy as jnp
from jax.experimental import pallas as pl

B, C, H, W = 128, 96, 32, 32
HW = H * W
BB = 2


def _tc_kernel(x_ref, row_ref, col_ref, out_ref):
    row = row_ref[:]
    col = col_ref[:]
    pos = (row[:, None, :] + col[None, :, :]).reshape(HW, C)
    for i in range(BB):
        out_ref[i] = x_ref[i].T + pos


def kernel(x, row_embed, col_embed):
    x3 = x.reshape(B, C, HW)
    out = pl.pallas_call(
        _tc_kernel,
        grid=(B // BB,),
        in_specs=[
            pl.BlockSpec((BB, C, HW), lambda b: (b, 0, 0)),
            pl.BlockSpec((H, C), lambda b: (0, 0)),
            pl.BlockSpec((W, C), lambda b: (0, 0)),
        ],
        out_specs=pl.BlockSpec((BB, HW, C), lambda b: (b, 0, 0)),
        out_shape=jax.ShapeDtypeStruct((B, HW, C), jnp.float32),
    )(x3, row_embed, col_embed)
    return out


# speedup vs baseline: 1.2066x; 1.2066x over previous
import jax
import jax.numpy as jnp
from jax.experimental import pallas as pl

B, C, H, W = 128, 96, 32, 32
HW = H * W
BB = 16


def _tc_kernel(x_ref, row_ref, col_ref, out_ref):
    row = row_ref[:]
    col = col_ref[:]
    pos = (row[:, None, :] + col[None, :, :]).reshape(HW, C)
    for i in range(BB):
        out_ref[i] = x_ref[i].T + pos


def kernel(x, row_embed, col_embed):
    x3 = x.reshape(B, C, HW)
    out = pl.pallas_call(
        _tc_kernel,
        grid=(B // BB,),
        in_specs=[
            pl.BlockSpec((BB, C, HW), lambda b: (b, 0, 0)),
            pl.BlockSpec((H, C), lambda b: (0, 0)),
            pl.BlockSpec((W, C), lambda b: (0, 0)),
        ],
        out_specs=pl.BlockSpec((BB, HW, C), lambda b: (b, 0, 0)),
        out_shape=jax.ShapeDtypeStruct((B, HW, C), jnp.float32),
    )(x3, row_embed, col_embed)
    return out
